# baseline (speedup 1.0000x reference)
import jax
import jax.numpy as jnp
from jax.experimental import pallas as pl
from jax.experimental.pallas import tpu as pltpu

_CHUNK_ROWS = 1024
_NBUF = 8
_DELTA = 4


def _relay_body(in_hbm, out_hbm, bufs, *sems):
    n_chunks = in_hbm.shape[0] // _CHUNK_ROWS
    in_sems, out_sems = sems[:_NBUF], sems[_NBUF:]
    ins, outs = [], []
    for i in range(n_chunks):
        s = i % _NBUF
        src = in_hbm.at[pl.ds(i * _CHUNK_ROWS, _CHUNK_ROWS)]
        dst = out_hbm.at[pl.ds(i * _CHUNK_ROWS, _CHUNK_ROWS)]
        ins.append(pltpu.make_async_copy(src, bufs.at[s], in_sems[s]))
        outs.append(pltpu.make_async_copy(bufs.at[s], dst, out_sems[s]))
    lead = _NBUF - _DELTA
    for i in range(min(lead, n_chunks)):
        ins[i].start()
    for i in range(n_chunks):
        j = i + lead
        if j < n_chunks:
            f = j - _NBUF
            if f >= 0:
                outs[f].wait()
            ins[j].start()
        ins[i].wait()
        outs[i].start()
    for i in range(max(0, n_chunks - _NBUF), n_chunks):
        outs[i].wait()


def kernel(image_token, text_cls, topk_idx, selected_pooled, is_rare, strength):
    B, N, D = image_token.shape
    rows = B * N
    x = image_token.reshape(rows, D)
    out = pl.pallas_call(
        _relay_body,
        out_shape=jax.ShapeDtypeStruct((rows, D), x.dtype),
        in_specs=[pl.BlockSpec(memory_space=pl.ANY)],
        out_specs=pl.BlockSpec(memory_space=pl.ANY),
        scratch_shapes=(
            [pltpu.VMEM((_NBUF, _CHUNK_ROWS, D), x.dtype)]
            + [pltpu.SemaphoreType.DMA] * (2 * _NBUF)
        ),
    )(x)
    return out.reshape(B, N, D)


# speedup vs baseline: 1.0046x; 1.0046x over previous
import jax
import jax.numpy as jnp
from jax.experimental import pallas as pl
from jax.experimental.pallas import tpu as pltpu

_BLOCK_ROWS = 4096


def _copy_body(in_ref, out_ref):
    out_ref[...] = in_ref[...]


def kernel(image_token, text_cls, topk_idx, selected_pooled, is_rare, strength):
    B, N, D = image_token.shape
    rows = B * N
    x = image_token.reshape(rows, D)
    out = pl.pallas_call(
        _copy_body,
        grid=(rows // _BLOCK_ROWS,),
        in_specs=[pl.BlockSpec((_BLOCK_ROWS, D), lambda i: (i, 0))],
        out_specs=pl.BlockSpec((_BLOCK_ROWS, D), lambda i: (i, 0)),
        out_shape=jax.ShapeDtypeStruct((rows, D), x.dtype),
        compiler_params=pltpu.CompilerParams(
            dimension_semantics=("parallel",),
            vmem_limit_bytes=100 * 1024 * 1024,
        ),
    )(x)
    return out.reshape(B, N, D)
